# baseline (speedup 1.0000x reference)
import functools

import jax
import jax.numpy as jnp
from jax import lax
from jax.experimental import pallas as pl
from jax.experimental.pallas import tpu as pltpu
from jax.experimental.pallas import tpu_sc as plsc

_N = 10000
_E = 320000
_D = 128
_BN_EPS = 1e-5

_NC = 2
_NS = 16
_NW = _NC * _NS
_ROWS = 320
_NPAD = _NW * _ROWS
_CAP = 16384
_CE = 4000
_RB = 128
_DUMMY = _ROWS
_CNTW = 336

_mesh = plsc.VectorSubcoreMesh(core_axis_name="c", subcore_axis_name="s")
_params = pltpu.CompilerParams(needs_layout_passes=False)

_DNUMS = lax.GatherDimensionNumbers(
    offset_dims=(), collapsed_slice_dims=(0,), start_index_map=(0,))


def _take16(v, idx):
    return lax.gather(v, idx[:, None], _DNUMS, (1,),
                      mode=lax.GatherScatterMode.PROMISE_IN_BOUNDS)


def _prep_body(src_hbm, dst_hbm, slists_hbm, llists_hbm, sizes_hbm, cnt_hbm,
               sstage0, dstage0, sstage1, dstage1, slist, llist, sizev, cntv,
               sem0, sem1):
    wid = lax.axis_index("s") * _NC + lax.axis_index("c")
    lo = wid * _ROWS
    lane15 = jnp.full((16,), 15, jnp.int32)

    def stage_start(ch, sst, dst_, sem):
        pltpu.async_copy(src_hbm.at[pl.ds(ch * _CE, _CE)], sst, sem)
        pltpu.async_copy(dst_hbm.at[pl.ds(ch * _CE, _CE)], dst_, sem)

    def stage_wait(ch, sst, dst_, sem):
        pltpu.make_async_copy(src_hbm.at[pl.ds(ch * _CE, _CE)], sst, sem).wait()
        pltpu.make_async_copy(dst_hbm.at[pl.ds(ch * _CE, _CE)], dst_, sem).wait()

    def scan_chunk(sstage, dstage, posv):
        def grp(g, posv):
            s16a = sstage[pl.ds(g * 32, 16)]
            d16a = dstage[pl.ds(g * 32, 16)]
            s16b = sstage[pl.ds(g * 32 + 16, 16)]
            d16b = dstage[pl.ds(g * 32 + 16, 16)]
            ma = (d16a >= lo) & (d16a < lo + _ROWS)
            mb = (d16b >= lo) & (d16b < lo + _ROWS)
            ca = plsc.cumsum(ma.astype(jnp.int32))
            cb = plsc.cumsum(mb.astype(jnp.int32))
            bca = _take16(ca, lane15)
            bcb = _take16(cb, lane15)
            tgta = posv + ca - 1
            posv2 = posv + bca
            tgtb = posv2 + cb - 1
            plsc.store_scatter(slist, [tgta], s16a, mask=ma)
            plsc.store_scatter(llist, [tgta], d16a - lo, mask=ma)
            plsc.store_scatter(slist, [tgtb], s16b, mask=mb)
            plsc.store_scatter(llist, [tgtb], d16b - lo, mask=mb)
            return posv2 + bcb

        return lax.fori_loop(0, _CE // 32, grp, posv)

    nch = _E // _CE

    def pair(p, posv):
        c0 = 2 * p
        stage_start(c0 + 1, sstage1, dstage1, sem1)
        stage_wait(c0, sstage0, dstage0, sem0)
        posv = scan_chunk(sstage0, dstage0, posv)

        @pl.when(p + 1 < nch // 2)
        def _():
            stage_start(c0 + 2, sstage0, dstage0, sem0)

        stage_wait(c0 + 1, sstage1, dstage1, sem1)
        return scan_chunk(sstage1, dstage1, posv)

    stage_start(0, sstage0, dstage0, sem0)
    posv = lax.fori_loop(0, nch // 2, pair, jnp.zeros((16,), jnp.int32))
    pos = posv[0]

    zeros16 = jnp.zeros((16,), jnp.int32)
    dummy16 = jnp.full((16,), _DUMMY, jnp.int32)
    for k in range(_RB // 16 + 2):
        slist[pl.ds(pos + k * 16, 16)] = zeros16
        llist[pl.ds(pos + k * 16, 16)] = dummy16

    zf = jnp.zeros((16,), jnp.float32)

    def zero_cnt(i, _):
        cntv[pl.ds(i * 16, 16)] = zf
        return 0

    lax.fori_loop(0, _CNTW // 16, zero_cnt, 0)

    ones16 = jnp.ones((16,), jnp.float32)

    def cnt_blk(g, _):
        lns = llist[pl.ds(g * 16, 16)]
        plsc.addupdate_scatter(cntv, [lns], ones16)
        return 0

    nblk = (pos + 15) // 16
    lax.fori_loop(0, nblk, cnt_blk, 0)

    sizev[...] = jnp.full((16,), pos, jnp.int32)
    pltpu.sync_copy(slist, slists_hbm.at[wid])
    pltpu.sync_copy(llist, llists_hbm.at[wid])
    pltpu.sync_copy(sizev, sizes_hbm.at[wid])
    pltpu.sync_copy(cntv.at[pl.ds(0, _ROWS)], cnt_hbm.at[pl.ds(lo, _ROWS)])


def _prep(src, dst):
    return pl.kernel(
        _prep_body,
        out_type=(
            jax.ShapeDtypeStruct((_NW, _CAP), jnp.int32),
            jax.ShapeDtypeStruct((_NW, _CAP), jnp.int32),
            jax.ShapeDtypeStruct((_NW, 16), jnp.int32),
            jax.ShapeDtypeStruct((_NPAD,), jnp.float32),
        ),
        mesh=_mesh,
        compiler_params=_params,
        scratch_types=[
            pltpu.VMEM((_CE,), jnp.int32),
            pltpu.VMEM((_CE,), jnp.int32),
            pltpu.VMEM((_CE,), jnp.int32),
            pltpu.VMEM((_CE,), jnp.int32),
            pltpu.VMEM((_CAP,), jnp.int32),
            pltpu.VMEM((_CAP,), jnp.int32),
            pltpu.VMEM((16,), jnp.int32),
            pltpu.VMEM((_CNTW,), jnp.float32),
            pltpu.SemaphoreType.DMA,
            pltpu.SemaphoreType.DMA,
        ],
    )(src, dst)


def _agg_body(table_hbm, slists_hbm, llists_hbm, sizes_hbm, agg_hbm,
              slist, llist, sizev, acc, rb0, rb1, rb2, sem0, sem1, sem2):
    wid = lax.axis_index("s") * _NC + lax.axis_index("c")
    lo = wid * _ROWS

    pltpu.async_copy(slists_hbm.at[wid], slist, sem0)
    pltpu.async_copy(llists_hbm.at[wid], llist, sem0)
    pltpu.async_copy(sizes_hbm.at[wid], sizev, sem0)
    pltpu.make_async_copy(slists_hbm.at[wid], slist, sem0).wait()
    pltpu.make_async_copy(llists_hbm.at[wid], llist, sem0).wait()
    pltpu.make_async_copy(sizes_hbm.at[wid], sizev, sem0).wait()
    size = sizev[...][0]

    zf = jnp.zeros((16,), jnp.float32)

    def zero_acc(i, _):
        acc[pl.ds(i * 16, 16)] = zf
        return 0

    lax.fori_loop(0, (_ROWS + 1) * _D // 16, zero_acc, 0)

    def start(ci, rb, sem):
        pltpu.async_copy(table_hbm.at[slist.at[pl.ds(ci * _RB, _RB)]], rb, sem)

    def wait(ci, rb, sem):
        pltpu.make_async_copy(table_hbm.at[slist.at[pl.ds(ci * _RB, _RB)]],
                              rb, sem).wait()

    iot = lax.iota(jnp.int32, 16)
    kio = [iot + k * 16 for k in range(_D // 16)]

    def acc_chunk(base, rb):
        def blk(b, _):
            lns = llist[pl.ds(base + b * 16, 16)]
            offs = lns * _D

            def edge_vals(j):
                oj = offs[j]
                vals = [rb[b * 16 + j, pl.ds(k * 16, 16)]
                        for k in range(_D // 16)]
                return oj, vals

            oj_p, vals_p = edge_vals(0)
            for j in range(1, 16):
                oj_n = offs[j]
                vals_n = []
                for k in range(_D // 16):
                    v = rb[b * 16 + j, pl.ds(k * 16, 16)]
                    plsc.addupdate(acc.at[pl.ds(oj_p + k * 16, 16)], vals_p[k])
                    vals_n.append(v)
                oj_p, vals_p = oj_n, vals_n
            for k in range(_D // 16):
                plsc.addupdate(acc.at[pl.ds(oj_p + k * 16, 16)], vals_p[k])
            return 0

        lax.fori_loop(0, _RB // 16, blk, 0)

    nchunks = (size + _RB - 1) // _RB
    bufs = ((rb0, sem0), (rb1, sem1), (rb2, sem2))

    for i, (rb, sem) in enumerate(bufs):
        @pl.when(i < nchunks)
        def _(i=i, rb=rb, sem=sem):
            start(i, rb, sem)

    def triple(t, _):
        for i, (rb, sem) in enumerate(bufs):
            c = 3 * t + i

            @pl.when(c < nchunks)
            def _(c=c, rb=rb, sem=sem):
                wait(c, rb, sem)
                acc_chunk(c * _RB, rb)

                @pl.when(c + 3 < nchunks)
                def _():
                    start(c + 3, rb, sem)

        return 0

    lax.fori_loop(0, (nchunks + 2) // 3, triple, 0)

    pltpu.sync_copy(acc.at[pl.ds(0, _ROWS * _D)],
                    agg_hbm.at[pl.ds(lo * _D, _ROWS * _D)])


def _agg(table, slists, llists, sizes):
    return pl.kernel(
        _agg_body,
        out_type=jax.ShapeDtypeStruct((_NPAD * _D,), jnp.float32),
        mesh=_mesh,
        compiler_params=_params,
        scratch_types=[
            pltpu.VMEM((_CAP,), jnp.int32),
            pltpu.VMEM((_CAP,), jnp.int32),
            pltpu.VMEM((16,), jnp.int32),
            pltpu.VMEM(((_ROWS + 1) * _D,), jnp.float32),
            pltpu.VMEM((_RB, _D), jnp.float32),
            pltpu.VMEM((_RB, _D), jnp.float32),
            pltpu.VMEM((_RB, _D), jnp.float32),
            pltpu.SemaphoreType.DMA,
            pltpu.SemaphoreType.DMA,
            pltpu.SemaphoreType.DMA,
        ],
    )(table, slists, llists, sizes)


def _dense_r_block(x_ref, Wr_ref, b_ref, o_ref):
    o_ref[...] = (jnp.dot(x_ref[...], Wr_ref[...],
                          preferred_element_type=jnp.float32) + b_ref[...])


def _dense_r(x, Wr, b):
    n = x.shape[0]
    R = 2000
    return pl.pallas_call(
        _dense_r_block,
        grid=(n // R,),
        in_specs=[
            pl.BlockSpec((R, _D), lambda i: (i, 0)),
            pl.BlockSpec((_D, _D), lambda i: (0, 0)),
            pl.BlockSpec((1, _D), lambda i: (0, 0)),
        ],
        out_specs=pl.BlockSpec((R, _D), lambda i: (i, 0)),
        out_shape=jax.ShapeDtypeStruct((n, _D), jnp.float32),
    )(x, Wr, b)


def _dense_l_block(relu, agg_ref, cnt_ref, r_ref, Wl_ref, scale_ref,
                   shift_ref, o_ref):
    cnt = jnp.maximum(cnt_ref[...], 1.0)
    mean = agg_ref[...] / cnt
    t = jnp.dot(mean, Wl_ref[...], preferred_element_type=jnp.float32)
    t = t + r_ref[...]
    t = t * scale_ref[...] + shift_ref[...]
    if relu:
        t = jnp.maximum(t, 0.0)
    o_ref[...] = t


def _dense_l(agg, cnt, r, Wl, scale, shift, relu):
    n = r.shape[0]
    R = 2000
    return pl.pallas_call(
        functools.partial(_dense_l_block, relu),
        grid=(n // R,),
        in_specs=[
            pl.BlockSpec((R, _D), lambda i: (i, 0)),
            pl.BlockSpec((R, 1), lambda i: (i, 0)),
            pl.BlockSpec((R, _D), lambda i: (i, 0)),
            pl.BlockSpec((_D, _D), lambda i: (0, 0)),
            pl.BlockSpec((1, _D), lambda i: (0, 0)),
            pl.BlockSpec((1, _D), lambda i: (0, 0)),
        ],
        out_specs=pl.BlockSpec((R, _D), lambda i: (i, 0)),
        out_shape=jax.ShapeDtypeStruct((n, _D), jnp.float32),
    )(agg, cnt, r, Wl, scale, shift)


def kernel(x, edge_index, W1l, b1l, W1r, bn_g, bn_b, bn_rm, bn_rv, W2l, b2l, W2r):
    src = edge_index[0]
    dst = edge_index[1]

    slists, llists, sizes, cnt_pad = _prep(src, dst)
    cnt2d = cnt_pad[:, None]

    r1 = _dense_r(x, W1r, b1l[None, :])
    agg1 = _agg(x, slists, llists, sizes).reshape(_NPAD, _D)

    scale = bn_g * jax.lax.rsqrt(bn_rv + _BN_EPS)
    shift = bn_b - bn_rm * scale
    h = _dense_l(agg1, cnt2d, r1, W1l, scale[None, :], shift[None, :], True)

    r2 = _dense_r(h, W2r, b2l[None, :])
    agg2 = _agg(h, slists, llists, sizes).reshape(_NPAD, _D)
    ones = jnp.ones((1, _D), jnp.float32)
    zeros = jnp.zeros((1, _D), jnp.float32)
    out = _dense_l(agg2, cnt2d, r2, W2l, ones, zeros, False)
    return out


# speedup vs baseline: 1.1297x; 1.1297x over previous
import functools

import jax
import jax.numpy as jnp
from jax import lax
from jax.experimental import pallas as pl
from jax.experimental.pallas import tpu as pltpu
from jax.experimental.pallas import tpu_sc as plsc

_N = 10000
_E = 320000
_D = 128
_BN_EPS = 1e-5

_NC = 2
_NS = 16
_NW = _NC * _NS
_ROWS = 320
_NPAD = _NW * _ROWS
_CAP = 16384
_CE = 3200
_RB = 128
_DUMMY = _ROWS
_CNTW = 336

_mesh = plsc.VectorSubcoreMesh(core_axis_name="c", subcore_axis_name="s")
_params = pltpu.CompilerParams(needs_layout_passes=False)

_DNUMS = lax.GatherDimensionNumbers(
    offset_dims=(), collapsed_slice_dims=(0,), start_index_map=(0,))


def _take16(v, idx):
    return lax.gather(v, idx[:, None], _DNUMS, (1,),
                      mode=lax.GatherScatterMode.PROMISE_IN_BOUNDS)


def _prep_body(src_hbm, dst_hbm, slists_hbm, llists_hbm, sizes_hbm, cnt_hbm,
               sstage0, dstage0, sstage1, dstage1, slist, llist, sizev, cntv,
               sem0, sem1):
    wid = lax.axis_index("s") * _NC + lax.axis_index("c")
    lo = wid * _ROWS
    lane15 = jnp.full((16,), 15, jnp.int32)

    def stage_start(ch, sst, dst_, sem):
        pltpu.async_copy(src_hbm.at[pl.ds(ch * _CE, _CE)], sst, sem)
        pltpu.async_copy(dst_hbm.at[pl.ds(ch * _CE, _CE)], dst_, sem)

    def stage_wait(ch, sst, dst_, sem):
        pltpu.make_async_copy(src_hbm.at[pl.ds(ch * _CE, _CE)], sst, sem).wait()
        pltpu.make_async_copy(dst_hbm.at[pl.ds(ch * _CE, _CE)], dst_, sem).wait()

    def scan_chunk(sstage, dstage, posv):
        def grp(g, posv):
            base = g * 64
            svs, dvs, ms, cs, bcs = [], [], [], [], []
            for q in range(4):
                s16 = sstage[pl.ds(base + q * 16, 16)]
                d16 = dstage[pl.ds(base + q * 16, 16)]
                m = (d16 >= lo) & (d16 < lo + _ROWS)
                c = plsc.cumsum(m.astype(jnp.int32))
                svs.append(s16)
                dvs.append(d16)
                ms.append(m)
                cs.append(c)
                bcs.append(_take16(c, lane15))
            off = posv
            for q in range(4):
                tgt = off + cs[q] - 1
                plsc.store_scatter(slist, [tgt], svs[q], mask=ms[q])
                plsc.store_scatter(llist, [tgt], dvs[q] - lo, mask=ms[q])
                off = off + bcs[q]
            return off

        return lax.fori_loop(0, _CE // 64, grp, posv)

    nch = _E // _CE

    def pair(p, posv):
        c0 = 2 * p
        stage_start(c0 + 1, sstage1, dstage1, sem1)
        stage_wait(c0, sstage0, dstage0, sem0)
        posv = scan_chunk(sstage0, dstage0, posv)

        @pl.when(p + 1 < nch // 2)
        def _():
            stage_start(c0 + 2, sstage0, dstage0, sem0)

        stage_wait(c0 + 1, sstage1, dstage1, sem1)
        return scan_chunk(sstage1, dstage1, posv)

    stage_start(0, sstage0, dstage0, sem0)
    posv = lax.fori_loop(0, nch // 2, pair, jnp.zeros((16,), jnp.int32))
    pos = posv[0]

    zeros16 = jnp.zeros((16,), jnp.int32)
    dummy16 = jnp.full((16,), _DUMMY, jnp.int32)
    for k in range(_RB // 16 + 2):
        slist[pl.ds(pos + k * 16, 16)] = zeros16
        llist[pl.ds(pos + k * 16, 16)] = dummy16

    zf = jnp.zeros((16,), jnp.float32)

    def zero_cnt(i, _):
        cntv[pl.ds(i * 16, 16)] = zf
        return 0

    lax.fori_loop(0, _CNTW // 16, zero_cnt, 0)

    ones16 = jnp.ones((16,), jnp.float32)

    def cnt_blk(g, _):
        lns = llist[pl.ds(g * 16, 16)]
        plsc.addupdate_scatter(cntv, [lns], ones16)
        return 0

    nblk = (pos + 15) // 16
    lax.fori_loop(0, nblk, cnt_blk, 0)

    sizev[...] = jnp.full((16,), pos, jnp.int32)
    pltpu.sync_copy(slist, slists_hbm.at[wid])
    pltpu.sync_copy(llist, llists_hbm.at[wid])
    pltpu.sync_copy(sizev, sizes_hbm.at[wid])
    pltpu.sync_copy(cntv.at[pl.ds(0, _ROWS)], cnt_hbm.at[pl.ds(lo, _ROWS)])


def _prep(src, dst):
    return pl.kernel(
        _prep_body,
        out_type=(
            jax.ShapeDtypeStruct((_NW, _CAP), jnp.int32),
            jax.ShapeDtypeStruct((_NW, _CAP), jnp.int32),
            jax.ShapeDtypeStruct((_NW, 16), jnp.int32),
            jax.ShapeDtypeStruct((_NPAD,), jnp.float32),
        ),
        mesh=_mesh,
        compiler_params=_params,
        scratch_types=[
            pltpu.VMEM((_CE,), jnp.int32),
            pltpu.VMEM((_CE,), jnp.int32),
            pltpu.VMEM((_CE,), jnp.int32),
            pltpu.VMEM((_CE,), jnp.int32),
            pltpu.VMEM((_CAP,), jnp.int32),
            pltpu.VMEM((_CAP,), jnp.int32),
            pltpu.VMEM((16,), jnp.int32),
            pltpu.VMEM((_CNTW,), jnp.float32),
            pltpu.SemaphoreType.DMA,
            pltpu.SemaphoreType.DMA,
        ],
    )(src, dst)


def _agg_body(table_hbm, slists_hbm, llists_hbm, sizes_hbm, agg_hbm,
              slist, llist, sizev, acc, rb0, rb1, sem0, sem1):
    wid = lax.axis_index("s") * _NC + lax.axis_index("c")
    lo = wid * _ROWS

    pltpu.async_copy(slists_hbm.at[wid], slist, sem0)
    pltpu.async_copy(llists_hbm.at[wid], llist, sem0)
    pltpu.async_copy(sizes_hbm.at[wid], sizev, sem0)
    pltpu.make_async_copy(slists_hbm.at[wid], slist, sem0).wait()
    pltpu.make_async_copy(llists_hbm.at[wid], llist, sem0).wait()
    pltpu.make_async_copy(sizes_hbm.at[wid], sizev, sem0).wait()
    size = sizev[...][0]

    zf = jnp.zeros((16,), jnp.float32)

    def zero_acc(i, _):
        acc[pl.ds(i * 16, 16)] = zf
        return 0

    lax.fori_loop(0, (_ROWS + 1) * _D // 16, zero_acc, 0)

    def start(ci, rb, sem):
        pltpu.async_copy(table_hbm.at[slist.at[pl.ds(ci * _RB, _RB)]], rb, sem)

    def wait(ci, rb, sem):
        pltpu.make_async_copy(table_hbm.at[slist.at[pl.ds(ci * _RB, _RB)]],
                              rb, sem).wait()

    iot = lax.iota(jnp.int32, 16)
    kio = [iot + k * 16 for k in range(_D // 16)]

    def acc_chunk(base, rb):
        def blk(b, _):
            lns = llist[pl.ds(base + b * 16, 16)]
            offs = lns * _D

            def edge_vals(j):
                oj = offs[j]
                vals = [rb[b * 16 + j, pl.ds(k * 16, 16)]
                        for k in range(_D // 16)]
                return oj, vals

            oj_p, vals_p = edge_vals(0)
            for j in range(1, 16):
                oj_n = offs[j]
                vals_n = []
                for k in range(_D // 16):
                    v = rb[b * 16 + j, pl.ds(k * 16, 16)]
                    plsc.addupdate(acc.at[pl.ds(oj_p + k * 16, 16)], vals_p[k])
                    vals_n.append(v)
                oj_p, vals_p = oj_n, vals_n
            for k in range(_D // 16):
                plsc.addupdate(acc.at[pl.ds(oj_p + k * 16, 16)], vals_p[k])
            return 0

        lax.fori_loop(0, _RB // 16, blk, 0)

    nchunks = (size + _RB - 1) // _RB

    @pl.when(nchunks > 0)
    def _():
        start(0, rb0, sem0)

    def pair(p, _):
        c0 = 2 * p
        c1 = c0 + 1

        @pl.when(c1 < nchunks)
        def _():
            start(c1, rb1, sem1)

        wait(c0, rb0, sem0)
        acc_chunk(c0 * _RB, rb0)

        @pl.when(c1 < nchunks)
        def _():
            @pl.when(c1 + 1 < nchunks)
            def _():
                start(c1 + 1, rb0, sem0)

            wait(c1, rb1, sem1)
            acc_chunk(c1 * _RB, rb1)

        return 0

    lax.fori_loop(0, (nchunks + 1) // 2, pair, 0)

    pltpu.sync_copy(acc.at[pl.ds(0, _ROWS * _D)],
                    agg_hbm.at[pl.ds(lo * _D, _ROWS * _D)])


def _agg(table, slists, llists, sizes):
    return pl.kernel(
        _agg_body,
        out_type=jax.ShapeDtypeStruct((_NPAD * _D,), jnp.float32),
        mesh=_mesh,
        compiler_params=_params,
        scratch_types=[
            pltpu.VMEM((_CAP,), jnp.int32),
            pltpu.VMEM((_CAP,), jnp.int32),
            pltpu.VMEM((16,), jnp.int32),
            pltpu.VMEM(((_ROWS + 1) * _D,), jnp.float32),
            pltpu.VMEM((_RB, _D), jnp.float32),
            pltpu.VMEM((_RB, _D), jnp.float32),
            pltpu.SemaphoreType.DMA,
            pltpu.SemaphoreType.DMA,
        ],
    )(table, slists, llists, sizes)


def _dense_r_block(x_ref, Wr_ref, b_ref, o_ref):
    o_ref[...] = (jnp.dot(x_ref[...], Wr_ref[...],
                          preferred_element_type=jnp.float32) + b_ref[...])


def _dense_r(x, Wr, b):
    n = x.shape[0]
    R = 2000
    return pl.pallas_call(
        _dense_r_block,
        grid=(n // R,),
        in_specs=[
            pl.BlockSpec((R, _D), lambda i: (i, 0)),
            pl.BlockSpec((_D, _D), lambda i: (0, 0)),
            pl.BlockSpec((1, _D), lambda i: (0, 0)),
        ],
        out_specs=pl.BlockSpec((R, _D), lambda i: (i, 0)),
        out_shape=jax.ShapeDtypeStruct((n, _D), jnp.float32),
    )(x, Wr, b)


def _dense_l_block(relu, agg_ref, cnt_ref, r_ref, Wl_ref, scale_ref,
                   shift_ref, o_ref):
    cnt = jnp.maximum(cnt_ref[...], 1.0)
    mean = agg_ref[...] / cnt
    t = jnp.dot(mean, Wl_ref[...], preferred_element_type=jnp.float32)
    t = t + r_ref[...]
    t = t * scale_ref[...] + shift_ref[...]
    if relu:
        t = jnp.maximum(t, 0.0)
    o_ref[...] = t


def _dense_l(agg, cnt, r, Wl, scale, shift, relu):
    n = r.shape[0]
    R = 2000
    return pl.pallas_call(
        functools.partial(_dense_l_block, relu),
        grid=(n // R,),
        in_specs=[
            pl.BlockSpec((R, _D), lambda i: (i, 0)),
            pl.BlockSpec((R, 1), lambda i: (i, 0)),
            pl.BlockSpec((R, _D), lambda i: (i, 0)),
            pl.BlockSpec((_D, _D), lambda i: (0, 0)),
            pl.BlockSpec((1, _D), lambda i: (0, 0)),
            pl.BlockSpec((1, _D), lambda i: (0, 0)),
        ],
        out_specs=pl.BlockSpec((R, _D), lambda i: (i, 0)),
        out_shape=jax.ShapeDtypeStruct((n, _D), jnp.float32),
    )(agg, cnt, r, Wl, scale, shift)


def kernel(x, edge_index, W1l, b1l, W1r, bn_g, bn_b, bn_rm, bn_rv, W2l, b2l, W2r):
    src = edge_index[0]
    dst = edge_index[1]

    slists, llists, sizes, cnt_pad = _prep(src, dst)
    cnt2d = cnt_pad[:, None]

    r1 = _dense_r(x, W1r, b1l[None, :])
    agg1 = _agg(x, slists, llists, sizes).reshape(_NPAD, _D)

    scale = bn_g * jax.lax.rsqrt(bn_rv + _BN_EPS)
    shift = bn_b - bn_rm * scale
    h = _dense_l(agg1, cnt2d, r1, W1l, scale[None, :], shift[None, :], True)

    r2 = _dense_r(h, W2r, b2l[None, :])
    agg2 = _agg(h, slists, llists, sizes).reshape(_NPAD, _D)
    ones = jnp.ones((1, _D), jnp.float32)
    zeros = jnp.zeros((1, _D), jnp.float32)
    out = _dense_l(agg2, cnt2d, r2, W2l, ones, zeros, False)
    return out
